# baseline (speedup 1.0000x reference)
import functools

import numpy as np

import jax
import jax.numpy as jnp
from jax import lax
from jax.experimental import pallas as pl
from jax.experimental.pallas import tpu as pltpu
from jax.experimental.pallas import tpu_sc as plsc

N = 10000
E = 320000
G = 64
D = 128
DO = 64
NP = 10240
NC, NS = 2, 16
NW = NC * NS
CHUNK = 128
CPW = 80
EP = NW * CPW * CHUNK
RPS = NP // NS
ACH = 64
APW = (EP // NW) // ACH
AHB = 32
NBUF = 4
NCHUNKS = NW * APW
RCH = (128,) * 5
RCHA = (64,) * 10
RB = 2560
NBLK = NP // RB

_f32 = jnp.float32
_bf16 = jnp.bfloat16
_PBLK = np.concatenate([np.arange(0, 32, 2), np.arange(1, 32, 2)])
_PERM = np.concatenate([32 * k + _PBLK for k in range(D // 32)])
_INV = np.argsort(_PERM)
_PMAT = np.zeros((D, D), np.float32)
_PMAT[_INV, np.arange(D)] = 1.0
_mesh = plsc.VectorSubcoreMesh(core_axis_name="c", subcore_axis_name="s",
                               num_cores=NC, num_subcores=NS)


def _deg_body(dst_hbm, out_hbm, idx_v, buf_v, acc_sh):
    c = lax.axis_index("c")
    s = lax.axis_index("s")
    w = s * NC + c

    def zrow(r, carry):
        for k in range(D // 16):
            buf_v[r, pl.ds(k * 16, 16)] = jnp.zeros((16,), _f32)
        return carry

    lax.fori_loop(0, CHUNK, zrow, 0)
    off = 0
    for nr in RCH:
        pltpu.sync_copy(buf_v.at[pl.ds(0, nr)],
                        acc_sh.at[pl.ds(s * RPS + off, nr)])
        off += nr

    def orow(r, carry):
        for k in range(D // 16):
            buf_v[r, pl.ds(k * 16, 16)] = jnp.ones((16,), _f32)
        return carry

    lax.fori_loop(0, CHUNK, orow, 0)
    pltpu.sync_copy(dst_hbm.at[pl.ds(w * CPW, CPW)], idx_v)
    plsc.subcore_barrier()

    def chunk(j, carry):
        pltpu.sync_copy(buf_v, acc_sh.at[idx_v.at[j]], add=True)
        return carry

    lax.fori_loop(0, CPW, chunk, 0)
    plsc.subcore_barrier()
    off = 0
    for nr in RCH:
        r0 = s * RPS + off
        pltpu.sync_copy(acc_sh.at[pl.ds(r0, nr)], out_hbm.at[c, pl.ds(r0, nr)])
        off += nr


_deg_call = pl.kernel(
    _deg_body,
    out_type=jax.ShapeDtypeStruct((NC, NP, D), _f32),
    mesh=_mesh,
    scratch_types=[
        pltpu.VMEM((CPW, CHUNK), jnp.int32),
        pltpu.VMEM((CHUNK, D), _f32),
        pltpu.VMEM_SHARED((NP, D), _f32),
    ],
)


def _agg_body(src_hbm, dst_hbm, y_hbm, out_hbm,
              idxs_v, idxd_v, g0, g1, g2, g3, f0, f1, acc_sh,
              s0, s1, s2, s3):
    c = lax.axis_index("c")
    s = lax.axis_index("s")
    w = s * NC + c
    gbufs = (g0, g1, g2, g3)
    fbufs = (f0, f1)
    sems = (s0, s1, s2, s3)

    def zrow(r, carry):
        for k in range(D // 16):
            f0[r, pl.ds(k * 16, 16)] = jnp.zeros((16,), _f32)
        return carry

    lax.fori_loop(0, ACH, zrow, 0)
    off = 0
    for nr in RCHA:
        pltpu.sync_copy(f0.at[pl.ds(0, nr)],
                        acc_sh.at[pl.ds(s * RPS + off, nr)])
        off += nr
    plsc.subcore_barrier()

    def half(h, carry):
        base = w * APW + h * AHB
        pltpu.sync_copy(src_hbm.at[pl.ds(base, AHB)], idxs_v)
        pltpu.sync_copy(dst_hbm.at[pl.ds(base, AHB)], idxd_v)
        pltpu.async_copy(y_hbm.at[idxs_v.at[0]], gbufs[0], sems[0])
        pltpu.async_copy(y_hbm.at[idxs_v.at[1]], gbufs[1], sems[1])

        def step(t, carry2):
            for b in range(NBUF):
                j = NBUF * t + b
                pltpu.make_async_copy(y_hbm.at[idxs_v.at[j]], gbufs[b],
                                      sems[b]).wait()
                jn = j + 2
                bn = (b + 2) % NBUF

                @pl.when(jn < AHB)
                def _():
                    pltpu.async_copy(y_hbm.at[idxs_v.at[jn]], gbufs[bn],
                                    sems[bn])

                gb = gbufs[b]
                fb = fbufs[b % 2]

                def crow(t2, carry3):
                    for dr in range(2):
                        r = 2 * t2 + dr
                        for k in range(D // 32):
                            v = gb[r, pl.ds(k * 16, 16)]
                            lo = lax.bitcast_convert_type(
                                lax.shift_left(v, 16), _f32)
                            hi = lax.bitcast_convert_type(
                                lax.bitwise_and(
                                    v, jnp.full((16,), -65536, jnp.int32)),
                                _f32)
                            fb[r, pl.ds(k * 32, 16)] = lo
                            fb[r, pl.ds(k * 32 + 16, 16)] = hi
                    return carry3

                lax.fori_loop(0, ACH // 2, crow, 0)
                pltpu.sync_copy(fb, acc_sh.at[idxd_v.at[j]], add=True)
            return carry2

        lax.fori_loop(0, AHB // NBUF, step, 0)
        return carry

    lax.fori_loop(0, APW // AHB, half, 0)
    plsc.subcore_barrier()
    off = 0
    for nr in RCH:
        rr = s * RPS + off
        pltpu.sync_copy(acc_sh.at[pl.ds(rr, nr)], out_hbm.at[c, pl.ds(rr, nr)])
        off += nr


_agg_call = pl.kernel(
    _agg_body,
    out_type=jax.ShapeDtypeStruct((NC, NP, D), _f32),
    mesh=_mesh,
    scratch_types=[
        pltpu.VMEM((AHB, ACH), jnp.int32),
        pltpu.VMEM((AHB, ACH), jnp.int32),
        pltpu.VMEM((ACH, D // 2), jnp.int32),
        pltpu.VMEM((ACH, D // 2), jnp.int32),
        pltpu.VMEM((ACH, D // 2), jnp.int32),
        pltpu.VMEM((ACH, D // 2), jnp.int32),
        pltpu.VMEM((ACH, D), _f32),
        pltpu.VMEM((ACH, D), _f32),
        pltpu.VMEM_SHARED((NP, D), _f32),
        pltpu.SemaphoreType.DMA,
        pltpu.SemaphoreType.DMA,
        pltpu.SemaphoreType.DMA,
        pltpu.SemaphoreType.DMA,
    ],
    compiler_params=pltpu.CompilerParams(use_tc_tiling_on_sc=False),
)


def _pre_body(x_ref, w_ref, d0_ref, d1_ref, y_ref, dv_ref):
    deg = d0_ref[:, 0:1] + d1_ref[:, 0:1] + 1.0
    dvb = jnp.broadcast_to(lax.rsqrt(deg), (RB, D))
    dv_ref[...] = dvb
    y_ref[...] = (jnp.dot(x_ref[...], w_ref[...],
                          preferred_element_type=_f32) * dvb).astype(_bf16)


def _mid_body(a0_ref, a1_ref, y1_ref, dv_ref, b_ref, w_ref, pm_ref, y2_ref):
    dv = dv_ref[...]
    ap = a0_ref[...] + a1_ref[...]
    acc = jnp.dot(ap, pm_ref[...], preferred_element_type=_f32) \
        + y1_ref[...].astype(_f32)
    t = jnp.maximum(acc * dv + b_ref[...], 0.0)
    y2_ref[...] = (jnp.dot(t, w_ref[...], preferred_element_type=_f32)
                   * dv).astype(_bf16)


def _post_body(a0_ref, a1_ref, y2_ref, dv_ref, b_ref, bt_ref, wfc_ref, bfc_ref,
               pm_ref, out_ref, ssum, smax, scnt):
    i = pl.program_id(0)

    @pl.when(i == 0)
    def _():
        ssum[...] = jnp.zeros_like(ssum)
        scnt[...] = jnp.zeros_like(scnt)
        smax[...] = jnp.full_like(smax, -jnp.inf)

    dv = dv_ref[...]
    ap = a0_ref[...] + a1_ref[...]
    acc = jnp.dot(ap, pm_ref[...], preferred_element_type=_f32) \
        + y2_ref[...].astype(_f32)
    t = jnp.maximum(acc * dv + b_ref[...], 0.0)
    bb = bt_ref[...]
    gi = lax.broadcasted_iota(jnp.int32, (RB, 128), 1)
    mask = (bb == gi).astype(_f32)
    dn = (((0,), (0,)), ((), ()))
    ssum[...] += lax.dot_general(mask, t, dn, preferred_element_type=_f32)
    scnt[...] += lax.dot_general(mask, jnp.ones((RB, D), _f32), dn,
                                 preferred_element_type=_f32)

    glo = jnp.min(bb)
    ghi = jnp.max(bb)

    def gbody(g, carry):
        m = bb == g
        v = jnp.where(m, t, -jnp.inf)
        mx = jnp.max(v, axis=0, keepdims=True)
        smax[pl.ds(g, 1), :] = jnp.maximum(smax[pl.ds(g, 1), :], mx)
        return carry

    lax.fori_loop(glo, ghi + 1, gbody, 0)

    @pl.when(i == NBLK - 1)
    def _():
        cnt = jnp.maximum(scnt[...], 1.0)
        mean = ssum[...] / cnt
        w = wfc_ref[...]
        o = (jnp.dot(mean[0:G, :], w[0:D, :], preferred_element_type=_f32)
             + jnp.dot(smax[0:G, :], w[D:2 * D, :], preferred_element_type=_f32)
             + jnp.dot(ssum[0:G, :], w[2 * D:3 * D, :],
                       preferred_element_type=_f32))
        out_ref[...] = o + bfc_ref[...]


_pre_call = pl.pallas_call(
    _pre_body,
    grid=(NBLK,),
    in_specs=[
        pl.BlockSpec((RB, D), lambda i: (i, 0)),
        pl.BlockSpec((D, D), lambda i: (0, 0)),
        pl.BlockSpec((RB, D), lambda i: (i, 0)),
        pl.BlockSpec((RB, D), lambda i: (i, 0)),
    ],
    out_specs=[
        pl.BlockSpec((RB, D), lambda i: (i, 0)),
        pl.BlockSpec((RB, D), lambda i: (i, 0)),
    ],
    out_shape=[
        jax.ShapeDtypeStruct((NP, D), _bf16),
        jax.ShapeDtypeStruct((NP, D), _f32),
    ],
)

_mid_call = pl.pallas_call(
    _mid_body,
    grid=(NBLK,),
    in_specs=[
        pl.BlockSpec((RB, D), lambda i: (i, 0)),
        pl.BlockSpec((RB, D), lambda i: (i, 0)),
        pl.BlockSpec((RB, D), lambda i: (i, 0)),
        pl.BlockSpec((RB, D), lambda i: (i, 0)),
        pl.BlockSpec((1, D), lambda i: (0, 0)),
        pl.BlockSpec((D, D), lambda i: (0, 0)),
        pl.BlockSpec((D, D), lambda i: (0, 0)),
    ],
    out_specs=pl.BlockSpec((RB, D), lambda i: (i, 0)),
    out_shape=jax.ShapeDtypeStruct((NP, D), _bf16),
)

_post_call = pl.pallas_call(
    _post_body,
    grid=(NBLK,),
    in_specs=[
        pl.BlockSpec((RB, D), lambda i: (i, 0)),
        pl.BlockSpec((RB, D), lambda i: (i, 0)),
        pl.BlockSpec((RB, D), lambda i: (i, 0)),
        pl.BlockSpec((RB, D), lambda i: (i, 0)),
        pl.BlockSpec((1, D), lambda i: (0, 0)),
        pl.BlockSpec((RB, 1), lambda i: (i, 0)),
        pl.BlockSpec((3 * D, DO), lambda i: (0, 0)),
        pl.BlockSpec((1, DO), lambda i: (0, 0)),
        pl.BlockSpec((D, D), lambda i: (0, 0)),
    ],
    out_specs=pl.BlockSpec((G, DO), lambda i: (0, 0)),
    out_shape=jax.ShapeDtypeStruct((G, DO), _f32),
    scratch_shapes=[
        pltpu.VMEM((128, D), _f32),
        pltpu.VMEM((128, D), _f32),
        pltpu.VMEM((128, D), _f32),
    ],
)


def _as_i32(yb):
    return jax.lax.bitcast_convert_type(
        yb.reshape(NP, D // 2, 2), jnp.int32)


def kernel(x, edge_index, batch, W1, b1, W2, b2, Wfc, bfc):
    src = edge_index[0]
    dst = edge_index[1]
    src_f = jnp.concatenate([src, jnp.zeros((EP - E,), jnp.int32)])
    dst_f = jnp.concatenate([dst, jnp.full((EP - E,), N, jnp.int32)])
    src_p = src_f.reshape(NCHUNKS, ACH)
    dst_p = dst_f.reshape(NCHUNKS, ACH)
    dst_deg = dst_f.reshape(NW * CPW, CHUNK)
    x_p = jnp.concatenate([x, jnp.zeros((NP - N, D), x.dtype)], axis=0)
    batch_p = jnp.concatenate(
        [batch, jnp.full((NP - N,), G, jnp.int32)])[:, None]

    deg = _deg_call(dst_deg)
    y1, dvb = _pre_call(x_p, W1, deg[0], deg[1])
    acc1 = _agg_call(src_p, dst_p, _as_i32(y1))
    pmat = jnp.asarray(_PMAT)
    y2 = _mid_call(acc1[0], acc1[1], y1, dvb, b1.reshape(1, D), W2, pmat)
    acc2 = _agg_call(src_p, dst_p, _as_i32(y2))
    return _post_call(acc2[0], acc2[1], y2, dvb, b2.reshape(1, D), batch_p,
                      Wfc, bfc.reshape(1, DO), pmat)


# speedup vs baseline: 1.0838x; 1.0838x over previous
import functools

import numpy as np

import jax
import jax.numpy as jnp
from jax import lax
from jax.experimental import pallas as pl
from jax.experimental.pallas import tpu as pltpu
from jax.experimental.pallas import tpu_sc as plsc

N = 10000
E = 320000
G = 64
D = 128
DO = 64
NP = 10240
NC, NS = 2, 16
NW = NC * NS
CHUNK = 128
CPW = 80
EP = NW * CPW * CHUNK
RPS = NP // NS
ACH = 64
APW = (EP // NW) // ACH
AHB = 32
NBUF = 4
NCHUNKS = NW * APW
RCH = (128,) * 5
RCHA = (64,) * 10
RB = 2560
NBLK = NP // RB

_f32 = jnp.float32
_bf16 = jnp.bfloat16
_PBLK = np.concatenate([np.arange(0, 32, 2), np.arange(1, 32, 2)])
_PERM = np.concatenate([32 * k + _PBLK for k in range(D // 32)])
_INV = np.argsort(_PERM)
_PMAT = np.zeros((D, D), np.float32)
_PMAT[_INV, np.arange(D)] = 1.0
_mesh = plsc.VectorSubcoreMesh(core_axis_name="c", subcore_axis_name="s",
                               num_cores=NC, num_subcores=NS)


def _deg_body(dst_hbm, out_hbm, idx_v, buf_v, acc_sh):
    c = lax.axis_index("c")
    s = lax.axis_index("s")
    w = s * NC + c

    def zrow(r, carry):
        buf_v[r, pl.ds(0, 16)] = jnp.zeros((16,), _f32)
        return carry

    lax.fori_loop(0, CHUNK, zrow, 0)
    off = 0
    for nr in RCH:
        pltpu.sync_copy(buf_v.at[pl.ds(0, nr)],
                        acc_sh.at[pl.ds(s * RPS + off, nr)])
        off += nr

    def orow(r, carry):
        buf_v[r, pl.ds(0, 16)] = jnp.ones((16,), _f32)
        return carry

    lax.fori_loop(0, CHUNK, orow, 0)
    pltpu.sync_copy(dst_hbm.at[pl.ds(w * CPW, CPW)], idx_v)
    plsc.subcore_barrier()

    def chunk(j, carry):
        pltpu.sync_copy(buf_v, acc_sh.at[idx_v.at[j]], add=True)
        return carry

    lax.fori_loop(0, CPW, chunk, 0)
    plsc.subcore_barrier()
    off = 0
    for nr in RCH:
        r0 = s * RPS + off
        pltpu.sync_copy(acc_sh.at[pl.ds(r0, nr)], out_hbm.at[c, pl.ds(r0, nr)])
        off += nr


_deg_call = pl.kernel(
    _deg_body,
    out_type=jax.ShapeDtypeStruct((NC, NP, 16), _f32),
    mesh=_mesh,
    scratch_types=[
        pltpu.VMEM((CPW, CHUNK), jnp.int32),
        pltpu.VMEM((CHUNK, 16), _f32),
        pltpu.VMEM_SHARED((NP, 16), _f32),
    ],
    compiler_params=pltpu.CompilerParams(use_tc_tiling_on_sc=False),
)


def _agg_body(src_hbm, dst_hbm, y_hbm, out_hbm,
              idxs_v, idxd_v, g0, g1, g2, g3, f0, f1, acc_sh,
              s0, s1, s2, s3):
    c = lax.axis_index("c")
    s = lax.axis_index("s")
    w = s * NC + c
    gbufs = (g0, g1, g2, g3)
    fbufs = (f0, f1)
    sems = (s0, s1, s2, s3)

    def zrow(r, carry):
        for k in range(D // 16):
            f0[r, pl.ds(k * 16, 16)] = jnp.zeros((16,), _f32)
        return carry

    lax.fori_loop(0, ACH, zrow, 0)
    off = 0
    for nr in RCHA:
        pltpu.sync_copy(f0.at[pl.ds(0, nr)],
                        acc_sh.at[pl.ds(s * RPS + off, nr)])
        off += nr
    plsc.subcore_barrier()

    def half(h, carry):
        base = w * APW + h * AHB
        pltpu.sync_copy(src_hbm.at[pl.ds(base, AHB)], idxs_v)
        pltpu.sync_copy(dst_hbm.at[pl.ds(base, AHB)], idxd_v)
        pltpu.async_copy(y_hbm.at[idxs_v.at[0]], gbufs[0], sems[0])
        pltpu.async_copy(y_hbm.at[idxs_v.at[1]], gbufs[1], sems[1])

        def step(t, carry2):
            for b in range(NBUF):
                j = NBUF * t + b
                pltpu.make_async_copy(y_hbm.at[idxs_v.at[j]], gbufs[b],
                                      sems[b]).wait()
                jn = j + 2
                bn = (b + 2) % NBUF

                @pl.when(jn < AHB)
                def _():
                    pltpu.async_copy(y_hbm.at[idxs_v.at[jn]], gbufs[bn],
                                    sems[bn])

                gb = gbufs[b]
                fb = fbufs[b % 2]

                def crow(r, carry3):
                    for k in range(D // 32):
                        v = gb[r, pl.ds(k * 16, 16)]
                        lo = lax.bitcast_convert_type(
                            lax.shift_left(v, 16), _f32)
                        hi = lax.bitcast_convert_type(
                            lax.bitwise_and(
                                v, jnp.full((16,), -65536, jnp.int32)), _f32)
                        fb[r, pl.ds(k * 32, 16)] = lo
                        fb[r, pl.ds(k * 32 + 16, 16)] = hi
                    return carry3

                lax.fori_loop(0, ACH, crow, 0)
                pltpu.sync_copy(fb, acc_sh.at[idxd_v.at[j]], add=True)
            return carry2

        lax.fori_loop(0, AHB // NBUF, step, 0)
        return carry

    lax.fori_loop(0, APW // AHB, half, 0)
    plsc.subcore_barrier()
    off = 0
    for nr in RCH:
        rr = s * RPS + off
        pltpu.sync_copy(acc_sh.at[pl.ds(rr, nr)], out_hbm.at[c, pl.ds(rr, nr)])
        off += nr


_agg_call = pl.kernel(
    _agg_body,
    out_type=jax.ShapeDtypeStruct((NC, NP, D), _f32),
    mesh=_mesh,
    scratch_types=[
        pltpu.VMEM((AHB, ACH), jnp.int32),
        pltpu.VMEM((AHB, ACH), jnp.int32),
        pltpu.VMEM((ACH, D // 2), jnp.int32),
        pltpu.VMEM((ACH, D // 2), jnp.int32),
        pltpu.VMEM((ACH, D // 2), jnp.int32),
        pltpu.VMEM((ACH, D // 2), jnp.int32),
        pltpu.VMEM((ACH, D), _f32),
        pltpu.VMEM((ACH, D), _f32),
        pltpu.VMEM_SHARED((NP, D), _f32),
        pltpu.SemaphoreType.DMA,
        pltpu.SemaphoreType.DMA,
        pltpu.SemaphoreType.DMA,
        pltpu.SemaphoreType.DMA,
    ],
    compiler_params=pltpu.CompilerParams(use_tc_tiling_on_sc=False),
)


def _pre_body(x_ref, w_ref, d0_ref, d1_ref, y_ref, dv_ref):
    deg = d0_ref[:, 0:1] + d1_ref[:, 0:1] + 1.0
    dvb = jnp.broadcast_to(lax.rsqrt(deg), (RB, D))
    dv_ref[...] = dvb
    y_ref[...] = (jnp.dot(x_ref[...], w_ref[...],
                          preferred_element_type=_f32) * dvb).astype(_bf16)


def _mid_body(a0_ref, a1_ref, y1_ref, dv_ref, b_ref, w_ref, pm_ref, y2_ref):
    dv = dv_ref[...]
    ap = a0_ref[...] + a1_ref[...]
    acc = jnp.dot(ap, pm_ref[...], preferred_element_type=_f32) \
        + y1_ref[...].astype(_f32)
    t = jnp.maximum(acc * dv + b_ref[...], 0.0)
    y2_ref[...] = (jnp.dot(t, w_ref[...], preferred_element_type=_f32)
                   * dv).astype(_bf16)


def _post_body(a0_ref, a1_ref, y2_ref, dv_ref, b_ref, bt_ref, wfc_ref, bfc_ref,
               pm_ref, out_ref, ssum, smax, scnt):
    i = pl.program_id(0)

    @pl.when(i == 0)
    def _():
        ssum[...] = jnp.zeros_like(ssum)
        scnt[...] = jnp.zeros_like(scnt)
        smax[...] = jnp.full_like(smax, -jnp.inf)

    dv = dv_ref[...]
    ap = a0_ref[...] + a1_ref[...]
    acc = jnp.dot(ap, pm_ref[...], preferred_element_type=_f32) \
        + y2_ref[...].astype(_f32)
    t = jnp.maximum(acc * dv + b_ref[...], 0.0)
    bb = bt_ref[...]
    gi = lax.broadcasted_iota(jnp.int32, (RB, 128), 1)
    mask = (bb == gi).astype(_f32)
    dn = (((0,), (0,)), ((), ()))
    ssum[...] += lax.dot_general(mask, t, dn, preferred_element_type=_f32)
    scnt[...] += lax.dot_general(mask, jnp.ones((RB, D), _f32), dn,
                                 preferred_element_type=_f32)

    glo = jnp.min(bb)
    ghi = jnp.max(bb)

    def gbody(g, carry):
        m = bb == g
        v = jnp.where(m, t, -jnp.inf)
        mx = jnp.max(v, axis=0, keepdims=True)
        smax[pl.ds(g, 1), :] = jnp.maximum(smax[pl.ds(g, 1), :], mx)
        return carry

    lax.fori_loop(glo, ghi + 1, gbody, 0)

    @pl.when(i == NBLK - 1)
    def _():
        cnt = jnp.maximum(scnt[...], 1.0)
        mean = ssum[...] / cnt
        w = wfc_ref[...]
        o = (jnp.dot(mean[0:G, :], w[0:D, :], preferred_element_type=_f32)
             + jnp.dot(smax[0:G, :], w[D:2 * D, :], preferred_element_type=_f32)
             + jnp.dot(ssum[0:G, :], w[2 * D:3 * D, :],
                       preferred_element_type=_f32))
        out_ref[...] = o + bfc_ref[...]


_pre_call = pl.pallas_call(
    _pre_body,
    grid=(NBLK,),
    in_specs=[
        pl.BlockSpec((RB, D), lambda i: (i, 0)),
        pl.BlockSpec((D, D), lambda i: (0, 0)),
        pl.BlockSpec((RB, 16), lambda i: (i, 0)),
        pl.BlockSpec((RB, 16), lambda i: (i, 0)),
    ],
    out_specs=[
        pl.BlockSpec((RB, D), lambda i: (i, 0)),
        pl.BlockSpec((RB, D), lambda i: (i, 0)),
    ],
    out_shape=[
        jax.ShapeDtypeStruct((NP, D), _bf16),
        jax.ShapeDtypeStruct((NP, D), _f32),
    ],
)

_mid_call = pl.pallas_call(
    _mid_body,
    grid=(NBLK,),
    in_specs=[
        pl.BlockSpec((RB, D), lambda i: (i, 0)),
        pl.BlockSpec((RB, D), lambda i: (i, 0)),
        pl.BlockSpec((RB, D), lambda i: (i, 0)),
        pl.BlockSpec((RB, D), lambda i: (i, 0)),
        pl.BlockSpec((1, D), lambda i: (0, 0)),
        pl.BlockSpec((D, D), lambda i: (0, 0)),
        pl.BlockSpec((D, D), lambda i: (0, 0)),
    ],
    out_specs=pl.BlockSpec((RB, D), lambda i: (i, 0)),
    out_shape=jax.ShapeDtypeStruct((NP, D), _bf16),
)

_post_call = pl.pallas_call(
    _post_body,
    grid=(NBLK,),
    in_specs=[
        pl.BlockSpec((RB, D), lambda i: (i, 0)),
        pl.BlockSpec((RB, D), lambda i: (i, 0)),
        pl.BlockSpec((RB, D), lambda i: (i, 0)),
        pl.BlockSpec((RB, D), lambda i: (i, 0)),
        pl.BlockSpec((1, D), lambda i: (0, 0)),
        pl.BlockSpec((RB, 1), lambda i: (i, 0)),
        pl.BlockSpec((3 * D, DO), lambda i: (0, 0)),
        pl.BlockSpec((1, DO), lambda i: (0, 0)),
        pl.BlockSpec((D, D), lambda i: (0, 0)),
    ],
    out_specs=pl.BlockSpec((G, DO), lambda i: (0, 0)),
    out_shape=jax.ShapeDtypeStruct((G, DO), _f32),
    scratch_shapes=[
        pltpu.VMEM((128, D), _f32),
        pltpu.VMEM((128, D), _f32),
        pltpu.VMEM((128, D), _f32),
    ],
)


def _as_i32(yb):
    return jax.lax.bitcast_convert_type(
        yb.reshape(NP, D // 2, 2), jnp.int32)


def kernel(x, edge_index, batch, W1, b1, W2, b2, Wfc, bfc):
    src = edge_index[0]
    dst = edge_index[1]
    src_f = jnp.concatenate([src, jnp.zeros((EP - E,), jnp.int32)])
    dst_f = jnp.concatenate([dst, jnp.full((EP - E,), N, jnp.int32)])
    src_p = src_f.reshape(NCHUNKS, ACH)
    dst_p = dst_f.reshape(NCHUNKS, ACH)
    dst_deg = dst_f.reshape(NW * CPW, CHUNK)
    x_p = jnp.concatenate([x, jnp.zeros((NP - N, D), x.dtype)], axis=0)
    batch_p = jnp.concatenate(
        [batch, jnp.full((NP - N,), G, jnp.int32)])[:, None]

    deg = _deg_call(dst_deg)
    y1, dvb = _pre_call(x_p, W1, deg[0], deg[1])
    acc1 = _agg_call(src_p, dst_p, _as_i32(y1))
    pmat = jnp.asarray(_PMAT)
    y2 = _mid_call(acc1[0], acc1[1], y1, dvb, b1.reshape(1, D), W2, pmat)
    acc2 = _agg_call(src_p, dst_p, _as_i32(y2))
    return _post_call(acc2[0], acc2[1], y2, dvb, b2.reshape(1, D), batch_p,
                      Wfc, bfc.reshape(1, DO), pmat)


# speedup vs baseline: 1.3043x; 1.2034x over previous
import functools

import numpy as np

import jax
import jax.numpy as jnp
from jax import lax
from jax.experimental import pallas as pl
from jax.experimental.pallas import tpu as pltpu
from jax.experimental.pallas import tpu_sc as plsc

N = 10000
E = 320000
G = 64
D = 128
DO = 64
NP = 10240
NC, NS = 2, 16
NW = NC * NS
CHUNK = 128
CPW = 80
EP = NW * CPW * CHUNK
RPS = NP // NS
ACH = 64
APW = (EP // NW) // ACH
AHB = 32
NBUF = 4
NCHUNKS = NW * APW
RCH = (128,) * 5
RCHA = (64,) * 10
RB = 2560
NBLK = NP // RB

_f32 = jnp.float32
_bf16 = jnp.bfloat16
_PBLK = np.concatenate([np.arange(0, 32, 2), np.arange(1, 32, 2)])
_PERM = np.concatenate([32 * k + _PBLK for k in range(D // 32)])
_INV = np.argsort(_PERM)
_PMAT = np.zeros((D, D), np.float32)
_PMAT[_INV, np.arange(D)] = 1.0
_mesh = plsc.VectorSubcoreMesh(core_axis_name="c", subcore_axis_name="s",
                               num_cores=NC, num_subcores=NS)


def _deg_body(dst_hbm, out_hbm, idx_v, buf_v, acc_sh):
    c = lax.axis_index("c")
    s = lax.axis_index("s")
    w = s * NC + c

    def zrow(r, carry):
        buf_v[r, pl.ds(0, 16)] = jnp.zeros((16,), _f32)
        return carry

    lax.fori_loop(0, CHUNK, zrow, 0)
    off = 0
    for nr in RCH:
        pltpu.sync_copy(buf_v.at[pl.ds(0, nr)],
                        acc_sh.at[pl.ds(s * RPS + off, nr)])
        off += nr

    def orow(r, carry):
        buf_v[r, pl.ds(0, 16)] = jnp.ones((16,), _f32)
        return carry

    lax.fori_loop(0, CHUNK, orow, 0)
    pltpu.sync_copy(dst_hbm.at[pl.ds(w * CPW, CPW)], idx_v)
    plsc.subcore_barrier()

    def chunk(j, carry):
        pltpu.sync_copy(buf_v, acc_sh.at[idx_v.at[j]], add=True)
        return carry

    lax.fori_loop(0, CPW, chunk, 0)
    plsc.subcore_barrier()
    off = 0
    for nr in RCH:
        r0 = s * RPS + off
        pltpu.sync_copy(acc_sh.at[pl.ds(r0, nr)], out_hbm.at[c, pl.ds(r0, nr)])
        off += nr


_deg_call = pl.kernel(
    _deg_body,
    out_type=jax.ShapeDtypeStruct((NC, NP, 16), _f32),
    mesh=_mesh,
    scratch_types=[
        pltpu.VMEM((CPW, CHUNK), jnp.int32),
        pltpu.VMEM((CHUNK, 16), _f32),
        pltpu.VMEM_SHARED((NP, 16), _f32),
    ],
    compiler_params=pltpu.CompilerParams(use_tc_tiling_on_sc=False),
)


def _agg_body(src_hbm, dst_hbm, y_hbm, out_hbm,
              idxs_v, idxd_v, g0, g1, g2, g3, acc_sh,
              s0, s1, s2, s3):
    c = lax.axis_index("c")
    s = lax.axis_index("s")
    w = s * NC + c
    gbufs = (g0, g1, g2, g3)
    sems = (s0, s1, s2, s3)

    def zrow(t2, carry):
        r = pl.multiple_of(2 * t2, 2)
        for k in range(D // 16):
            g0[pl.ds(r, 2), pl.ds(k * 16, 16)] = jnp.zeros((2, 16), _bf16)
        return carry

    lax.fori_loop(0, ACH // 2, zrow, 0)
    off = 0
    for nr in RCHA:
        pltpu.sync_copy(g0.at[pl.ds(0, nr)],
                        acc_sh.at[pl.ds(s * RPS + off, nr)])
        off += nr
    plsc.subcore_barrier()

    def half(h, carry):
        base = w * APW + h * AHB
        pltpu.sync_copy(src_hbm.at[pl.ds(base, AHB)], idxs_v)
        pltpu.sync_copy(dst_hbm.at[pl.ds(base, AHB)], idxd_v)
        pltpu.async_copy(y_hbm.at[idxs_v.at[0]], gbufs[0], sems[0])
        pltpu.async_copy(y_hbm.at[idxs_v.at[1]], gbufs[1], sems[1])

        def step(t, carry2):
            for b in range(NBUF):
                j = NBUF * t + b
                pltpu.make_async_copy(y_hbm.at[idxs_v.at[j]], gbufs[b],
                                      sems[b]).wait()
                jn = j + 2
                bn = (b + 2) % NBUF

                @pl.when(jn < AHB)
                def _():
                    pltpu.async_copy(y_hbm.at[idxs_v.at[jn]], gbufs[bn],
                                    sems[bn])

                pltpu.sync_copy(gbufs[b], acc_sh.at[idxd_v.at[j]], add=True)
            return carry2

        lax.fori_loop(0, AHB // NBUF, step, 0)
        return carry

    lax.fori_loop(0, APW // AHB, half, 0)
    plsc.subcore_barrier()
    off = 0
    for nr in RCH:
        rr = s * RPS + off
        pltpu.sync_copy(acc_sh.at[pl.ds(rr, nr)], out_hbm.at[c, pl.ds(rr, nr)])
        off += nr


_agg_call = pl.kernel(
    _agg_body,
    out_type=jax.ShapeDtypeStruct((NC, NP, D), _bf16),
    mesh=_mesh,
    scratch_types=[
        pltpu.VMEM((AHB, ACH), jnp.int32),
        pltpu.VMEM((AHB, ACH), jnp.int32),
        pltpu.VMEM((ACH, D), _bf16),
        pltpu.VMEM((ACH, D), _bf16),
        pltpu.VMEM((ACH, D), _bf16),
        pltpu.VMEM((ACH, D), _bf16),
        pltpu.VMEM_SHARED((NP, D), _bf16),
        pltpu.SemaphoreType.DMA,
        pltpu.SemaphoreType.DMA,
        pltpu.SemaphoreType.DMA,
        pltpu.SemaphoreType.DMA,
    ],
    compiler_params=pltpu.CompilerParams(use_tc_tiling_on_sc=False),
)


def _pre_body(x_ref, w_ref, d0_ref, d1_ref, y_ref, dv_ref):
    deg = d0_ref[:, 0:1] + d1_ref[:, 0:1] + 1.0
    dvb = jnp.broadcast_to(lax.rsqrt(deg), (RB, D))
    dv_ref[...] = dvb
    y_ref[...] = (jnp.dot(x_ref[...], w_ref[...],
                          preferred_element_type=_f32) * dvb).astype(_bf16)


def _mid_body(a0_ref, a1_ref, y1_ref, dv_ref, b_ref, w_ref, y2_ref):
    dv = dv_ref[...]
    acc = (a0_ref[...].astype(_f32) + a1_ref[...].astype(_f32)
           + y1_ref[...].astype(_f32))
    t = jnp.maximum(acc * dv + b_ref[...], 0.0)
    y2_ref[...] = (jnp.dot(t, w_ref[...], preferred_element_type=_f32)
                   * dv).astype(_bf16)


def _post_body(a0_ref, a1_ref, y2_ref, dv_ref, b_ref, bt_ref, wfc_ref, bfc_ref,
               out_ref, ssum, smax, scnt):
    i = pl.program_id(0)

    @pl.when(i == 0)
    def _():
        ssum[...] = jnp.zeros_like(ssum)
        scnt[...] = jnp.zeros_like(scnt)
        smax[...] = jnp.full_like(smax, -jnp.inf)

    dv = dv_ref[...]
    acc = (a0_ref[...].astype(_f32) + a1_ref[...].astype(_f32)
           + y2_ref[...].astype(_f32))
    t = jnp.maximum(acc * dv + b_ref[...], 0.0)
    bb = bt_ref[...]
    gi = lax.broadcasted_iota(jnp.int32, (RB, 128), 1)
    mask = (bb == gi).astype(_f32)
    dn = (((0,), (0,)), ((), ()))
    ssum[...] += lax.dot_general(mask, t, dn, preferred_element_type=_f32)
    scnt[...] += lax.dot_general(mask, jnp.ones((RB, D), _f32), dn,
                                 preferred_element_type=_f32)

    glo = jnp.min(bb)
    ghi = jnp.max(bb)

    def gbody(g, carry):
        m = bb == g
        v = jnp.where(m, t, -jnp.inf)
        mx = jnp.max(v, axis=0, keepdims=True)
        smax[pl.ds(g, 1), :] = jnp.maximum(smax[pl.ds(g, 1), :], mx)
        return carry

    lax.fori_loop(glo, ghi + 1, gbody, 0)

    @pl.when(i == NBLK - 1)
    def _():
        cnt = jnp.maximum(scnt[...], 1.0)
        mean = ssum[...] / cnt
        w = wfc_ref[...]
        o = (jnp.dot(mean[0:G, :], w[0:D, :], preferred_element_type=_f32)
             + jnp.dot(smax[0:G, :], w[D:2 * D, :], preferred_element_type=_f32)
             + jnp.dot(ssum[0:G, :], w[2 * D:3 * D, :],
                       preferred_element_type=_f32))
        out_ref[...] = o + bfc_ref[...]


_pre_call = pl.pallas_call(
    _pre_body,
    grid=(NBLK,),
    in_specs=[
        pl.BlockSpec((RB, D), lambda i: (i, 0)),
        pl.BlockSpec((D, D), lambda i: (0, 0)),
        pl.BlockSpec((RB, 16), lambda i: (i, 0)),
        pl.BlockSpec((RB, 16), lambda i: (i, 0)),
    ],
    out_specs=[
        pl.BlockSpec((RB, D), lambda i: (i, 0)),
        pl.BlockSpec((RB, D), lambda i: (i, 0)),
    ],
    out_shape=[
        jax.ShapeDtypeStruct((NP, D), _bf16),
        jax.ShapeDtypeStruct((NP, D), _f32),
    ],
)

_mid_call = pl.pallas_call(
    _mid_body,
    grid=(NBLK,),
    in_specs=[
        pl.BlockSpec((RB, D), lambda i: (i, 0)),
        pl.BlockSpec((RB, D), lambda i: (i, 0)),
        pl.BlockSpec((RB, D), lambda i: (i, 0)),
        pl.BlockSpec((RB, D), lambda i: (i, 0)),
        pl.BlockSpec((1, D), lambda i: (0, 0)),
        pl.BlockSpec((D, D), lambda i: (0, 0)),
    ],
    out_specs=pl.BlockSpec((RB, D), lambda i: (i, 0)),
    out_shape=jax.ShapeDtypeStruct((NP, D), _bf16),
)

_post_call = pl.pallas_call(
    _post_body,
    grid=(NBLK,),
    in_specs=[
        pl.BlockSpec((RB, D), lambda i: (i, 0)),
        pl.BlockSpec((RB, D), lambda i: (i, 0)),
        pl.BlockSpec((RB, D), lambda i: (i, 0)),
        pl.BlockSpec((RB, D), lambda i: (i, 0)),
        pl.BlockSpec((1, D), lambda i: (0, 0)),
        pl.BlockSpec((RB, 1), lambda i: (i, 0)),
        pl.BlockSpec((3 * D, DO), lambda i: (0, 0)),
        pl.BlockSpec((1, DO), lambda i: (0, 0)),
    ],
    out_specs=pl.BlockSpec((G, DO), lambda i: (0, 0)),
    out_shape=jax.ShapeDtypeStruct((G, DO), _f32),
    scratch_shapes=[
        pltpu.VMEM((128, D), _f32),
        pltpu.VMEM((128, D), _f32),
        pltpu.VMEM((128, D), _f32),
    ],
)


def _as_i32(yb):
    return jax.lax.bitcast_convert_type(
        yb.reshape(NP, D // 2, 2), jnp.int32)


def kernel(x, edge_index, batch, W1, b1, W2, b2, Wfc, bfc):
    src = edge_index[0]
    dst = edge_index[1]
    src_f = jnp.concatenate([src, jnp.zeros((EP - E,), jnp.int32)])
    dst_f = jnp.concatenate([dst, jnp.full((EP - E,), N, jnp.int32)])
    src_p = src_f.reshape(NCHUNKS, ACH)
    dst_p = dst_f.reshape(NCHUNKS, ACH)
    dst_deg = dst_f.reshape(NW * CPW, CHUNK)
    x_p = jnp.concatenate([x, jnp.zeros((NP - N, D), x.dtype)], axis=0)
    batch_p = jnp.concatenate(
        [batch, jnp.full((NP - N,), G, jnp.int32)])[:, None]

    deg = _deg_call(dst_deg)
    y1, dvb = _pre_call(x_p, W1, deg[0], deg[1])
    acc1 = _agg_call(src_p, dst_p, y1)
    y2 = _mid_call(acc1[0], acc1[1], y1, dvb, b1.reshape(1, D), W2)
    acc2 = _agg_call(src_p, dst_p, y2)
    return _post_call(acc2[0], acc2[1], y2, dvb, b2.reshape(1, D), batch_p,
                      Wfc, bfc.reshape(1, DO))


# speedup vs baseline: 1.3055x; 1.0010x over previous
import jax
import jax.numpy as jnp
from jax import lax
from jax.experimental import pallas as pl
from jax.experimental.pallas import tpu as pltpu
from jax.experimental.pallas import tpu_sc as plsc

N = 10000
E = 320000
G = 64
D = 128
DO = 64
NP = 10240
NC, NS = 2, 16
NW = NC * NS
CHUNK = 128
CPW = 80
EP = NW * CPW * CHUNK
RPS = NP // NS
ACH = 64
APW = (EP // NW) // ACH
AHB = 32
NBUF = 4
NCHUNKS = NW * APW
RCH = (128,) * 5
RCHA = (64,) * 10
RB = 2560
NBLK = NP // RB

_f32 = jnp.float32
_bf16 = jnp.bfloat16
_mesh = plsc.VectorSubcoreMesh(core_axis_name="c", subcore_axis_name="s",
                               num_cores=NC, num_subcores=NS)


def _deg_body(dst_hbm, out_hbm, idx_v, buf_v, acc_sh):
    c = lax.axis_index("c")
    s = lax.axis_index("s")
    w = s * NC + c

    def zrow(r, carry):
        buf_v[r, pl.ds(0, 16)] = jnp.zeros((16,), _f32)
        return carry

    lax.fori_loop(0, CHUNK, zrow, 0)
    off = 0
    for nr in RCH:
        pltpu.sync_copy(buf_v.at[pl.ds(0, nr)],
                        acc_sh.at[pl.ds(s * RPS + off, nr)])
        off += nr

    def orow(r, carry):
        buf_v[r, pl.ds(0, 16)] = jnp.ones((16,), _f32)
        return carry

    lax.fori_loop(0, CHUNK, orow, 0)
    pltpu.sync_copy(dst_hbm.at[pl.ds(w * CPW, CPW)], idx_v)
    plsc.subcore_barrier()

    def chunk(j, carry):
        pltpu.sync_copy(buf_v, acc_sh.at[idx_v.at[j]], add=True)
        return carry

    lax.fori_loop(0, CPW, chunk, 0)
    plsc.subcore_barrier()
    off = 0
    for nr in RCH:
        r0 = s * RPS + off
        pltpu.sync_copy(acc_sh.at[pl.ds(r0, nr)], out_hbm.at[c, pl.ds(r0, nr)])
        off += nr


_deg_call = pl.kernel(
    _deg_body,
    out_type=jax.ShapeDtypeStruct((NC, NP, 16), _f32),
    mesh=_mesh,
    scratch_types=[
        pltpu.VMEM((CPW, CHUNK), jnp.int32),
        pltpu.VMEM((CHUNK, 16), _f32),
        pltpu.VMEM_SHARED((NP, 16), _f32),
    ],
    compiler_params=pltpu.CompilerParams(use_tc_tiling_on_sc=False),
)


def _agg_body(src_hbm, dst_hbm, y_hbm, out_hbm,
              idxs_v, idxd_v, g0, g1, g2, g3, acc_sh,
              s0, s1, s2, s3):
    c = lax.axis_index("c")
    s = lax.axis_index("s")
    w = s * NC + c
    gbufs = (g0, g1, g2, g3)
    sems = (s0, s1, s2, s3)

    def zrow(t2, carry):
        r = pl.multiple_of(2 * t2, 2)
        for k in range(D // 16):
            g0[pl.ds(r, 2), pl.ds(k * 16, 16)] = jnp.zeros((2, 16), _bf16)
        return carry

    lax.fori_loop(0, ACH // 2, zrow, 0)
    off = 0
    for nr in RCHA:
        pltpu.sync_copy(g0.at[pl.ds(0, nr)],
                        acc_sh.at[pl.ds(s * RPS + off, nr)])
        off += nr
    plsc.subcore_barrier()

    def half(h, carry):
        base = w * APW + h * AHB
        pltpu.sync_copy(src_hbm.at[pl.ds(base, AHB)], idxs_v)
        pltpu.sync_copy(dst_hbm.at[pl.ds(base, AHB)], idxd_v)
        pltpu.async_copy(y_hbm.at[idxs_v.at[0]], gbufs[0], sems[0])
        pltpu.async_copy(y_hbm.at[idxs_v.at[1]], gbufs[1], sems[1])

        def step(t, carry2):
            for b in range(NBUF):
                j = NBUF * t + b
                pltpu.make_async_copy(y_hbm.at[idxs_v.at[j]], gbufs[b],
                                      sems[b]).wait()
                jn = j + 2
                bn = (b + 2) % NBUF

                @pl.when(jn < AHB)
                def _():
                    pltpu.async_copy(y_hbm.at[idxs_v.at[jn]], gbufs[bn],
                                    sems[bn])

                pltpu.sync_copy(gbufs[b], acc_sh.at[idxd_v.at[j]], add=True)
            return carry2

        lax.fori_loop(0, AHB // NBUF, step, 0)
        return carry

    lax.fori_loop(0, APW // AHB, half, 0)
    plsc.subcore_barrier()
    off = 0
    for nr in RCH:
        rr = s * RPS + off
        pltpu.sync_copy(acc_sh.at[pl.ds(rr, nr)], out_hbm.at[c, pl.ds(rr, nr)])
        off += nr


_agg_call = pl.kernel(
    _agg_body,
    out_type=jax.ShapeDtypeStruct((NC, NP, D), _bf16),
    mesh=_mesh,
    scratch_types=[
        pltpu.VMEM((AHB, ACH), jnp.int32),
        pltpu.VMEM((AHB, ACH), jnp.int32),
        pltpu.VMEM((ACH, D), _bf16),
        pltpu.VMEM((ACH, D), _bf16),
        pltpu.VMEM((ACH, D), _bf16),
        pltpu.VMEM((ACH, D), _bf16),
        pltpu.VMEM_SHARED((NP, D), _bf16),
        pltpu.SemaphoreType.DMA,
        pltpu.SemaphoreType.DMA,
        pltpu.SemaphoreType.DMA,
        pltpu.SemaphoreType.DMA,
    ],
    compiler_params=pltpu.CompilerParams(use_tc_tiling_on_sc=False),
)


def _pre_body(x_ref, w_ref, d0_ref, d1_ref, y_ref, dv_ref):
    deg = d0_ref[:, 0:1] + d1_ref[:, 0:1] + 1.0
    dvb = jnp.broadcast_to(lax.rsqrt(deg), (RB, D))
    dv_ref[...] = dvb
    y_ref[...] = (jnp.dot(x_ref[...], w_ref[...],
                          preferred_element_type=_f32) * dvb).astype(_bf16)


def _mid_body(a0_ref, a1_ref, y1_ref, dv_ref, b_ref, w_ref, y2_ref):
    dv = dv_ref[...]
    acc = (a0_ref[...].astype(_f32) + a1_ref[...].astype(_f32)
           + y1_ref[...].astype(_f32))
    t = jnp.maximum(acc * dv + b_ref[...], 0.0)
    y2_ref[...] = (jnp.dot(t, w_ref[...], preferred_element_type=_f32)
                   * dv).astype(_bf16)


def _post_body(a0_ref, a1_ref, y2_ref, dv_ref, b_ref, bt_ref, wfc_ref, bfc_ref,
               out_ref, ssum, smax, scnt):
    i = pl.program_id(0)

    @pl.when(i == 0)
    def _():
        ssum[...] = jnp.zeros_like(ssum)
        scnt[...] = jnp.zeros_like(scnt)
        smax[...] = jnp.full_like(smax, -jnp.inf)

    dv = dv_ref[...]
    acc = (a0_ref[...].astype(_f32) + a1_ref[...].astype(_f32)
           + y2_ref[...].astype(_f32))
    t = jnp.maximum(acc * dv + b_ref[...], 0.0)
    bb = bt_ref[...]
    gi = lax.broadcasted_iota(jnp.int32, (RB, 128), 1)
    mask = (bb == gi).astype(_f32)
    dn = (((0,), (0,)), ((), ()))
    ssum[...] += lax.dot_general(mask, t, dn, preferred_element_type=_f32)
    scnt[...] += lax.dot_general(mask, jnp.ones((RB, D), _f32), dn,
                                 preferred_element_type=_f32)

    glo = jnp.min(bb)
    ghi = jnp.max(bb)

    def gbody(g, carry):
        m = bb == g
        v = jnp.where(m, t, -jnp.inf)
        mx = jnp.max(v, axis=0, keepdims=True)
        smax[pl.ds(g, 1), :] = jnp.maximum(smax[pl.ds(g, 1), :], mx)
        return carry

    lax.fori_loop(glo, ghi + 1, gbody, 0)

    @pl.when(i == NBLK - 1)
    def _():
        cnt = jnp.maximum(scnt[...], 1.0)
        mean = ssum[...] / cnt
        w = wfc_ref[...]
        o = (jnp.dot(mean[0:G, :], w[0:D, :], preferred_element_type=_f32)
             + jnp.dot(smax[0:G, :], w[D:2 * D, :], preferred_element_type=_f32)
             + jnp.dot(ssum[0:G, :], w[2 * D:3 * D, :],
                       preferred_element_type=_f32))
        out_ref[...] = o + bfc_ref[...]


_pre_call = pl.pallas_call(
    _pre_body,
    grid=(NBLK,),
    in_specs=[
        pl.BlockSpec((RB, D), lambda i: (i, 0)),
        pl.BlockSpec((D, D), lambda i: (0, 0)),
        pl.BlockSpec((RB, 16), lambda i: (i, 0)),
        pl.BlockSpec((RB, 16), lambda i: (i, 0)),
    ],
    out_specs=[
        pl.BlockSpec((RB, D), lambda i: (i, 0)),
        pl.BlockSpec((RB, D), lambda i: (i, 0)),
    ],
    out_shape=[
        jax.ShapeDtypeStruct((NP, D), _bf16),
        jax.ShapeDtypeStruct((NP, D), _f32),
    ],
)

_mid_call = pl.pallas_call(
    _mid_body,
    grid=(NBLK,),
    in_specs=[
        pl.BlockSpec((RB, D), lambda i: (i, 0)),
        pl.BlockSpec((RB, D), lambda i: (i, 0)),
        pl.BlockSpec((RB, D), lambda i: (i, 0)),
        pl.BlockSpec((RB, D), lambda i: (i, 0)),
        pl.BlockSpec((1, D), lambda i: (0, 0)),
        pl.BlockSpec((D, D), lambda i: (0, 0)),
    ],
    out_specs=pl.BlockSpec((RB, D), lambda i: (i, 0)),
    out_shape=jax.ShapeDtypeStruct((NP, D), _bf16),
)

_post_call = pl.pallas_call(
    _post_body,
    grid=(NBLK,),
    in_specs=[
        pl.BlockSpec((RB, D), lambda i: (i, 0)),
        pl.BlockSpec((RB, D), lambda i: (i, 0)),
        pl.BlockSpec((RB, D), lambda i: (i, 0)),
        pl.BlockSpec((RB, D), lambda i: (i, 0)),
        pl.BlockSpec((1, D), lambda i: (0, 0)),
        pl.BlockSpec((RB, 1), lambda i: (i, 0)),
        pl.BlockSpec((3 * D, DO), lambda i: (0, 0)),
        pl.BlockSpec((1, DO), lambda i: (0, 0)),
    ],
    out_specs=pl.BlockSpec((G, DO), lambda i: (0, 0)),
    out_shape=jax.ShapeDtypeStruct((G, DO), _f32),
    scratch_shapes=[
        pltpu.VMEM((128, D), _f32),
        pltpu.VMEM((128, D), _f32),
        pltpu.VMEM((128, D), _f32),
    ],
)


def kernel(x, edge_index, batch, W1, b1, W2, b2, Wfc, bfc):
    src = edge_index[0]
    dst = edge_index[1]
    src_f = jnp.concatenate([src, jnp.zeros((EP - E,), jnp.int32)])
    dst_f = jnp.concatenate([dst, jnp.full((EP - E,), N, jnp.int32)])
    src_p = src_f.reshape(NCHUNKS, ACH)
    dst_p = dst_f.reshape(NCHUNKS, ACH)
    dst_deg = dst_f.reshape(NW * CPW, CHUNK)
    x_p = jnp.concatenate([x, jnp.zeros((NP - N, D), x.dtype)], axis=0)
    batch_p = jnp.concatenate(
        [batch, jnp.full((NP - N,), G, jnp.int32)])[:, None]

    deg = _deg_call(dst_deg)
    y1, dvb = _pre_call(x_p, W1, deg[0], deg[1])
    acc1 = _agg_call(src_p, dst_p, y1)
    y2 = _mid_call(acc1[0], acc1[1], y1, dvb, b1.reshape(1, D), W2)
    acc2 = _agg_call(src_p, dst_p, y2)
    return _post_call(acc2[0], acc2[1], y2, dvb, b2.reshape(1, D), batch_p,
                      Wfc, bfc.reshape(1, DO))
